# baseline (speedup 1.0000x reference)
import functools

import jax
import jax.numpy as jnp
from jax import lax
from jax.experimental import pallas as pl
from jax.experimental.pallas import tpu as pltpu
from jax.experimental.pallas import tpu_sc as plsc

N_NODES = 10000
N_EDGES = 320000
N_GRAPHS = 128
D_IN = 128
D_EMB = 256

_NS = 16
_CH = 128
_NB = 3
_ZROWS = 640
_ACC_ROWS = _NS * _ZROWS
_TRASH = N_NODES


def _seg_sum_sc(D2, colsplit):
    mesh = plsc.VectorSubcoreMesh(core_axis_name="c", subcore_axis_name="s")
    ept = (N_EDGES if colsplit else N_EDGES // 2) // _NS
    nfull = ept // _CH
    tail = ept - nfull * _CH
    assert nfull % 2 == 0

    @functools.partial(
        pl.kernel,
        out_type=jax.ShapeDtypeStruct((2, _ACC_ROWS, D2), jnp.float32),
        mesh=mesh,
        scratch_types=[
            [pltpu.VMEM((_CH,), jnp.int32) for _ in range(2)],
            [pltpu.VMEM((_CH,), jnp.int32) for _ in range(2)],
            [pltpu.VMEM((_CH,), jnp.int32) for _ in range(2)],
            [pltpu.VMEM((_CH, D2), jnp.float32) for _ in range(2)],
            pltpu.VMEM_SHARED((_ACC_ROWS, D2), jnp.float32),
            pltpu.SemaphoreType.DMA,
            pltpu.SemaphoreType.DMA,
            pltpu.SemaphoreType.DMA,
        ],
    )
    def k(src_hbm, dst_hbm, table_hbm, out_hbm,
          lsrc, ldst, sdst, rows, acc, sem_l, sem_g, sem_s):
        c = lax.axis_index("c")
        s = lax.axis_index("s")
        if colsplit:
            ebase = s * ept
        else:
            ebase = c * (N_EDGES // 2) + s * ept

        zero16 = jnp.zeros((16,), jnp.float32)

        def zrow(r, carry):
            for g in range(D2 // 16):
                rows[0][r, pl.ds(g * 16, 16)] = zero16
            return carry

        lax.fori_loop(0, _CH, zrow, 0)
        zb = s * _ZROWS
        for kk in range(_ZROWS // _CH):
            pltpu.sync_copy(rows[0], acc.at[pl.ds(zb + kk * _CH, _CH)])
        plsc.subcore_barrier()

        def lissue(j, b):
            base = ebase + j * _CH
            pltpu.async_copy(src_hbm.at[pl.ds(base, _CH)], lsrc[b], sem_l)
            pltpu.async_copy(dst_hbm.at[pl.ds(base, _CH)], ldst[b], sem_l)

        def lwait(j, b):
            base = ebase + j * _CH
            pltpu.make_async_copy(
                src_hbm.at[pl.ds(base, _CH)], lsrc[b], sem_l).wait()
            pltpu.make_async_copy(
                dst_hbm.at[pl.ds(base, _CH)], ldst[b], sem_l).wait()

        def to_idx(b):
            if colsplit:
                for g in range(_CH // 16):
                    sl = pl.ds(g * 16, 16)
                    lsrc[b][sl] = lsrc[b][sl] * 2 + c

        def gissue(b):
            pltpu.async_copy(table_hbm.at[lsrc[b]], rows[b], sem_g)

        def gwait(b):
            pltpu.make_async_copy(
                table_hbm.at[lsrc[b]], rows[b], sem_g).wait()

        def dstcopy(b):
            for g in range(_CH // 16):
                sl = pl.ds(g * 16, 16)
                sdst[b][sl] = ldst[b][sl]

        def sissue(b):
            pltpu.async_copy(rows[b], acc.at[sdst[b]], sem_s, add=True)

        def swait(b):
            pltpu.make_async_copy(rows[b], acc.at[sdst[b]], sem_s).wait()

        def body(j, b, first, last):
            gwait(b)
            dstcopy(b)
            sissue(b)
            if not last:
                @pl.when(j + 2 < nfull)
                def _():
                    lissue(j + 2, b)

                lwait(j + 1, 1 - b)
                to_idx(1 - b)
                if not first:
                    swait(1 - b)
                gissue(1 - b)

        lissue(0, 0)
        lwait(0, 0)
        to_idx(0)
        gissue(0)
        lissue(1, 1)
        body(0, 0, True, False)

        def pair(jp, carry):
            j = 2 * jp
            body(j + 1, 1, False, False)
            body(j + 2, 0, False, False)
            return carry

        lax.fori_loop(0, nfull // 2 - 1, pair, 0)
        body(nfull - 1, 1, False, True)
        swait(0)
        swait(1)

        if tail:
            for g in range(_CH // 16):
                sl = pl.ds(g * 16, 16)
                lsrc[0][sl] = jnp.zeros((16,), jnp.int32)
                sdst[0][sl] = jnp.full((16,), _TRASH, jnp.int32)
            tb = ebase + nfull * _CH
            pltpu.sync_copy(src_hbm.at[pl.ds(tb, tail)],
                            lsrc[0].at[pl.ds(0, tail)])
            pltpu.sync_copy(dst_hbm.at[pl.ds(tb, tail)],
                            sdst[0].at[pl.ds(0, tail)])
            to_idx(0)
            pltpu.async_copy(table_hbm.at[lsrc[0]], rows[0], sem_g).wait()
            pltpu.sync_copy(rows[0], acc.at[sdst[0]], add=True)

        plsc.subcore_barrier()

        for kk in range(_ZROWS // _CH):
            pltpu.sync_copy(acc.at[pl.ds(zb + kk * _CH, _CH)],
                            out_hbm.at[c, pl.ds(zb + kk * _CH, _CH)])

    return k


_BN = 1000


def _mlp_body(colsplit, h_ref, a_ref, wa_ref, ba_ref, wb_ref, bb_ref, o_ref):
    if colsplit:
        agg = jnp.concatenate([a_ref[0], a_ref[1]], axis=1)
    else:
        agg = a_ref[0] + a_ref[1]
    z = h_ref[...] + agg
    t = jnp.maximum(
        jnp.dot(z, wa_ref[...], preferred_element_type=jnp.float32)
        + ba_ref[...], 0.0)
    o = (jnp.dot(t, wb_ref[...], preferred_element_type=jnp.float32)
         + bb_ref[...])
    o_ref[...] = jnp.maximum(o, 0.0)


def _mlp_tc(h, agg, Wa, ba, Wb, bb, colsplit):
    Din = h.shape[1]
    D2 = agg.shape[2]
    grid = (N_NODES // _BN,)
    return pl.pallas_call(
        functools.partial(_mlp_body, colsplit),
        grid=grid,
        in_specs=[
            pl.BlockSpec((_BN, Din), lambda i: (i, 0)),
            pl.BlockSpec((2, _BN, D2), lambda i: (0, i, 0)),
            pl.BlockSpec((Din, D_EMB), lambda i: (0, 0)),
            pl.BlockSpec((1, D_EMB), lambda i: (0, 0)),
            pl.BlockSpec((D_EMB, D_EMB), lambda i: (0, 0)),
            pl.BlockSpec((1, D_EMB), lambda i: (0, 0)),
        ],
        out_specs=pl.BlockSpec((_BN, D_EMB), lambda i: (i, 0)),
        out_shape=jax.ShapeDtypeStruct((N_NODES, D_EMB), jnp.float32),
    )(h, agg, Wa, ba.reshape(1, -1), Wb, bb.reshape(1, -1))


def _pool_body(ne_ref, batch_ref, o_ref, acc, cnt):
    i = pl.program_id(0)
    oh = (lax.broadcasted_iota(jnp.int32, (N_GRAPHS, _BN), 0)
          == batch_ref[0]).astype(jnp.float32)
    part = jnp.dot(oh, ne_ref[...], preferred_element_type=jnp.float32)
    pcnt = jnp.sum(oh, axis=1, keepdims=True)

    @pl.when(i == 0)
    def _():
        acc[...] = part
        cnt[...] = pcnt

    @pl.when(i > 0)
    def _():
        acc[...] += part
        cnt[...] += pcnt

    @pl.when(i == pl.num_programs(0) - 1)
    def _():
        o_ref[...] = acc[...] / jnp.maximum(cnt[...], 1.0)


def _pool_tc(h, batch3d):
    D = h.shape[1]
    return pl.pallas_call(
        _pool_body,
        grid=(N_NODES // _BN,),
        in_specs=[
            pl.BlockSpec((_BN, D), lambda i: (i, 0)),
            pl.BlockSpec((1, 1, _BN), lambda i: (i, 0, 0)),
        ],
        out_specs=pl.BlockSpec((N_GRAPHS, D), lambda i: (0, 0)),
        out_shape=jax.ShapeDtypeStruct((N_GRAPHS, D), jnp.float32),
        scratch_shapes=[
            pltpu.VMEM((N_GRAPHS, D), jnp.float32),
            pltpu.VMEM((N_GRAPHS, 1), jnp.float32),
        ],
    )(h, batch3d)


def kernel(x, edge_index, batch,
           W0a, b0a, W0b, b0b, W1a, b1a, W1b, b1b, W2a, b2a, W2b, b2b):
    params = [(W0a, b0a, W0b, b0b), (W1a, b1a, W1b, b1b),
              (W2a, b2a, W2b, b2b)]
    h = x
    hs = []
    pooled = []
    src = edge_index[0]
    dst = edge_index[1]
    batch3d = batch.reshape(N_NODES // _BN, 1, _BN)
    first = True
    for Wa, ba, Wb, bb in params:
        if first:
            agg = _seg_sum_sc(h.shape[1], False)(src, dst, h)
        else:
            D2 = h.shape[1] // 2
            table = h.reshape(2 * N_NODES, D2)
            agg = _seg_sum_sc(D2, True)(src, dst, table)
        h = _mlp_tc(h, agg, Wa, ba, Wb, bb, colsplit=not first)
        first = False
        hs.append(h)
        pooled.append(_pool_tc(h, batch3d))
    node_embed = jnp.concatenate(hs, axis=1)
    graph_embed = jnp.concatenate(pooled, axis=1)
    return graph_embed, node_embed


# speedup vs baseline: 1.1973x; 1.1973x over previous
import functools

import jax
import jax.numpy as jnp
from jax import lax
from jax.experimental import pallas as pl
from jax.experimental.pallas import tpu as pltpu
from jax.experimental.pallas import tpu_sc as plsc

N_NODES = 10000
N_EDGES = 320000
N_GRAPHS = 128
D_IN = 128
D_EMB = 256

_NS = 16
_CH = 128
_ACC_ROWS = 10016
_TRASH = N_NODES


def _seg_sum_sc(D2, colsplit):
    mesh = plsc.VectorSubcoreMesh(core_axis_name="c", subcore_axis_name="s")
    ept = (N_EDGES if colsplit else N_EDGES // 2) // _NS
    nfull = ept // _CH
    tail = ept - nfull * _CH
    assert nfull % 3 == 0

    @functools.partial(
        pl.kernel,
        out_type=jax.ShapeDtypeStruct((2, _ACC_ROWS, D2), jnp.float32),
        mesh=mesh,
        scratch_types=[
            [pltpu.VMEM((_CH,), jnp.int32) for _ in range(3)],
            [pltpu.VMEM((_CH,), jnp.int32) for _ in range(3)],
            [pltpu.VMEM((_CH,), jnp.int32) for _ in range(3)],
            [pltpu.VMEM((_CH, D2), jnp.float32) for _ in range(3)],
            pltpu.VMEM_SHARED((_ACC_ROWS, D2), jnp.float32),
            pltpu.SemaphoreType.DMA,
            pltpu.SemaphoreType.DMA,
            pltpu.SemaphoreType.DMA,
        ],
    )
    def k(src_hbm, dst_hbm, table_hbm, out_hbm,
          lsrc, ldst, sdst, rows, acc, sem_l, sem_g, sem_s):
        c = lax.axis_index("c")
        s = lax.axis_index("s")
        if colsplit:
            ebase = s * ept
        else:
            ebase = c * (N_EDGES // 2) + s * ept

        zero16 = jnp.zeros((16,), jnp.float32)

        def zrow(r, carry):
            for g in range(D2 // 16):
                rows[0][r, pl.ds(g * 16, 16)] = zero16
            return carry

        lax.fori_loop(0, _CH, zrow, 0)
        nzfull = _ACC_ROWS // _CH
        for kk in range(nzfull // _NS + 1):
            cid = s + kk * _NS

            @pl.when(cid < nzfull)
            def _():
                pltpu.sync_copy(rows[0], acc.at[pl.ds(cid * _CH, _CH)])

            @pl.when(cid == nzfull)
            def _():
                pltpu.sync_copy(rows[0].at[pl.ds(0, _ACC_ROWS - nzfull * _CH)],
                                acc.at[pl.ds(nzfull * _CH,
                                             _ACC_ROWS - nzfull * _CH)])

        plsc.subcore_barrier()

        def lissue(j, b):
            base = ebase + j * _CH
            pltpu.async_copy(src_hbm.at[pl.ds(base, _CH)], lsrc[b], sem_l)
            pltpu.async_copy(dst_hbm.at[pl.ds(base, _CH)], ldst[b], sem_l)

        def lwait(j, b):
            base = ebase + j * _CH
            pltpu.make_async_copy(
                src_hbm.at[pl.ds(base, _CH)], lsrc[b], sem_l).wait()
            pltpu.make_async_copy(
                dst_hbm.at[pl.ds(base, _CH)], ldst[b], sem_l).wait()

        def to_idx(b):
            if colsplit:
                for g in range(_CH // 16):
                    sl = pl.ds(g * 16, 16)
                    lsrc[b][sl] = lsrc[b][sl] * 2 + c

        def gissue(b):
            pltpu.async_copy(table_hbm.at[lsrc[b]], rows[b], sem_g)

        def gwait(b):
            pltpu.make_async_copy(
                table_hbm.at[lsrc[b]], rows[b], sem_g).wait()

        def dstcopy(b):
            for g in range(_CH // 16):
                sl = pl.ds(g * 16, 16)
                sdst[b][sl] = ldst[b][sl]

        def sissue(b):
            pltpu.async_copy(rows[b], acc.at[sdst[b]], sem_s, add=True)

        def swait(b):
            pltpu.make_async_copy(rows[b], acc.at[sdst[b]], sem_s).wait()

        def body(j, b):
            gwait(b)
            dstcopy(b)
            sissue(b)

            @pl.when(j + 3 < nfull)
            def _():
                lissue(j + 3, b)

            @pl.when(j + 2 < nfull)
            def _():
                b2 = (b + 2) % 3
                lwait(j + 2, b2)
                to_idx(b2)

                @pl.when(j > 0)
                def _():
                    swait(b2)

                gissue(b2)

        lissue(0, 0)
        lwait(0, 0)
        to_idx(0)
        gissue(0)
        lissue(1, 1)
        lwait(1, 1)
        to_idx(1)
        gissue(1)
        lissue(2, 2)

        def group(jp, carry):
            j = 3 * jp
            body(j, 0)
            body(j + 1, 1)
            body(j + 2, 2)
            return carry

        lax.fori_loop(0, nfull // 3, group, 0)
        swait(0)
        swait(1)
        swait(2)

        if tail:
            for g in range(_CH // 16):
                sl = pl.ds(g * 16, 16)
                lsrc[0][sl] = jnp.zeros((16,), jnp.int32)
                sdst[0][sl] = jnp.full((16,), _TRASH, jnp.int32)
            tb = ebase + nfull * _CH
            pltpu.sync_copy(src_hbm.at[pl.ds(tb, tail)],
                            lsrc[0].at[pl.ds(0, tail)])
            pltpu.sync_copy(dst_hbm.at[pl.ds(tb, tail)],
                            sdst[0].at[pl.ds(0, tail)])
            to_idx(0)
            pltpu.async_copy(table_hbm.at[lsrc[0]], rows[0], sem_g).wait()
            pltpu.sync_copy(rows[0], acc.at[sdst[0]], add=True)

        plsc.subcore_barrier()

        for kk in range(nzfull // _NS + 1):
            cid = s + kk * _NS

            @pl.when(cid < nzfull)
            def _():
                pltpu.sync_copy(acc.at[pl.ds(cid * _CH, _CH)],
                                out_hbm.at[c, pl.ds(cid * _CH, _CH)])

            @pl.when(cid == nzfull)
            def _():
                rem = _ACC_ROWS - nzfull * _CH
                pltpu.sync_copy(acc.at[pl.ds(nzfull * _CH, rem)],
                                out_hbm.at[c, pl.ds(nzfull * _CH, rem)])

    return k


_BN = 1000


def _mlp_body(colsplit, h_ref, a_ref, wa_ref, ba_ref, wb_ref, bb_ref, o_ref):
    if colsplit:
        agg = jnp.concatenate([a_ref[0], a_ref[1]], axis=1)
    else:
        agg = a_ref[0] + a_ref[1]
    z = h_ref[...] + agg
    t = jnp.maximum(
        jnp.dot(z, wa_ref[...], preferred_element_type=jnp.float32)
        + ba_ref[...], 0.0)
    o = (jnp.dot(t, wb_ref[...], preferred_element_type=jnp.float32)
         + bb_ref[...])
    o_ref[...] = jnp.maximum(o, 0.0)


def _mlp_tc(h, agg, Wa, ba, Wb, bb, colsplit):
    Din = h.shape[1]
    D2 = agg.shape[2]
    grid = (N_NODES // _BN,)
    return pl.pallas_call(
        functools.partial(_mlp_body, colsplit),
        grid=grid,
        in_specs=[
            pl.BlockSpec((_BN, Din), lambda i: (i, 0)),
            pl.BlockSpec((2, _BN, D2), lambda i: (0, i, 0)),
            pl.BlockSpec((Din, D_EMB), lambda i: (0, 0)),
            pl.BlockSpec((1, D_EMB), lambda i: (0, 0)),
            pl.BlockSpec((D_EMB, D_EMB), lambda i: (0, 0)),
            pl.BlockSpec((1, D_EMB), lambda i: (0, 0)),
        ],
        out_specs=pl.BlockSpec((_BN, D_EMB), lambda i: (i, 0)),
        out_shape=jax.ShapeDtypeStruct((N_NODES, D_EMB), jnp.float32),
    )(h, agg, Wa, ba.reshape(1, -1), Wb, bb.reshape(1, -1))


def _pool_body(ne_ref, batch_ref, o_ref, acc, cnt):
    i = pl.program_id(0)
    oh = (lax.broadcasted_iota(jnp.int32, (N_GRAPHS, _BN), 0)
          == batch_ref[0]).astype(jnp.float32)
    part = jnp.dot(oh, ne_ref[...], preferred_element_type=jnp.float32)
    pcnt = jnp.sum(oh, axis=1, keepdims=True)

    @pl.when(i == 0)
    def _():
        acc[...] = part
        cnt[...] = pcnt

    @pl.when(i > 0)
    def _():
        acc[...] += part
        cnt[...] += pcnt

    @pl.when(i == pl.num_programs(0) - 1)
    def _():
        o_ref[...] = acc[...] / jnp.maximum(cnt[...], 1.0)


def _pool_tc(h, batch3d):
    D = h.shape[1]
    return pl.pallas_call(
        _pool_body,
        grid=(N_NODES // _BN,),
        in_specs=[
            pl.BlockSpec((_BN, D), lambda i: (i, 0)),
            pl.BlockSpec((1, 1, _BN), lambda i: (i, 0, 0)),
        ],
        out_specs=pl.BlockSpec((N_GRAPHS, D), lambda i: (0, 0)),
        out_shape=jax.ShapeDtypeStruct((N_GRAPHS, D), jnp.float32),
        scratch_shapes=[
            pltpu.VMEM((N_GRAPHS, D), jnp.float32),
            pltpu.VMEM((N_GRAPHS, 1), jnp.float32),
        ],
    )(h, batch3d)


def kernel(x, edge_index, batch,
           W0a, b0a, W0b, b0b, W1a, b1a, W1b, b1b, W2a, b2a, W2b, b2b):
    params = [(W0a, b0a, W0b, b0b), (W1a, b1a, W1b, b1b),
              (W2a, b2a, W2b, b2b)]
    h = x
    hs = []
    pooled = []
    src = edge_index[0]
    dst = edge_index[1]
    batch3d = batch.reshape(N_NODES // _BN, 1, _BN)
    first = True
    for Wa, ba, Wb, bb in params:
        if first:
            agg = _seg_sum_sc(h.shape[1], False)(src, dst, h)
        else:
            D2 = h.shape[1] // 2
            table = h.reshape(2 * N_NODES, D2)
            agg = _seg_sum_sc(D2, True)(src, dst, table)
        h = _mlp_tc(h, agg, Wa, ba, Wb, bb, colsplit=not first)
        first = False
        hs.append(h)
        pooled.append(_pool_tc(h, batch3d))
    node_embed = jnp.concatenate(hs, axis=1)
    graph_embed = jnp.concatenate(pooled, axis=1)
    return graph_embed, node_embed


# speedup vs baseline: 1.3308x; 1.1115x over previous
import functools

import jax
import jax.numpy as jnp
from jax import lax
from jax.experimental import pallas as pl
from jax.experimental.pallas import tpu as pltpu
from jax.experimental.pallas import tpu_sc as plsc

N_NODES = 10000
N_EDGES = 320000
N_GRAPHS = 128
D_IN = 128
D_EMB = 256

_NS = 16
_CH = 96
_NB = 4
_ACC_ROWS = 10016
_TRASH = N_NODES


def _seg_sum_sc(D2, colsplit):
    mesh = plsc.VectorSubcoreMesh(core_axis_name="c", subcore_axis_name="s")
    ept = (N_EDGES if colsplit else N_EDGES // 2) // _NS
    nfull = ept // _CH
    tail = ept - nfull * _CH
    assert nfull % _NB == 0

    @functools.partial(
        pl.kernel,
        out_type=jax.ShapeDtypeStruct((2, _ACC_ROWS, D2), jnp.float32),
        mesh=mesh,
        scratch_types=[
            [pltpu.VMEM((_CH,), jnp.int32) for _ in range(_NB)],
            [pltpu.VMEM((_CH,), jnp.int32) for _ in range(_NB)],
            [pltpu.VMEM((_CH,), jnp.int32) for _ in range(_NB)],
            [pltpu.VMEM((_CH, D2), jnp.float32) for _ in range(_NB)],
            pltpu.VMEM_SHARED((_ACC_ROWS, D2), jnp.float32),
            pltpu.SemaphoreType.DMA,
            pltpu.SemaphoreType.DMA,
            pltpu.SemaphoreType.DMA,
        ],
    )
    def k(src_hbm, dst_hbm, table_hbm, out_hbm,
          lsrc, ldst, sdst, rows, acc, sem_l, sem_g, sem_s):
        c = lax.axis_index("c")
        s = lax.axis_index("s")
        if colsplit:
            ebase = s * ept
        else:
            ebase = c * (N_EDGES // 2) + s * ept

        zero16 = jnp.zeros((16,), jnp.float32)

        def zrow(r, carry):
            for g in range(D2 // 16):
                rows[0][r, pl.ds(g * 16, 16)] = zero16
            return carry

        lax.fori_loop(0, _CH, zrow, 0)
        nzfull = _ACC_ROWS // _CH
        for kk in range(nzfull // _NS + 1):
            cid = s + kk * _NS

            @pl.when(cid < nzfull)
            def _():
                pltpu.sync_copy(rows[0], acc.at[pl.ds(cid * _CH, _CH)])

            @pl.when(cid == nzfull)
            def _():
                pltpu.sync_copy(rows[0].at[pl.ds(0, _ACC_ROWS - nzfull * _CH)],
                                acc.at[pl.ds(nzfull * _CH,
                                             _ACC_ROWS - nzfull * _CH)])

        plsc.subcore_barrier()

        def lissue(j, b):
            base = ebase + j * _CH
            pltpu.async_copy(src_hbm.at[pl.ds(base, _CH)], lsrc[b], sem_l)
            pltpu.async_copy(dst_hbm.at[pl.ds(base, _CH)], ldst[b], sem_l)

        def lwait(j, b):
            base = ebase + j * _CH
            pltpu.make_async_copy(
                src_hbm.at[pl.ds(base, _CH)], lsrc[b], sem_l).wait()
            pltpu.make_async_copy(
                dst_hbm.at[pl.ds(base, _CH)], ldst[b], sem_l).wait()

        def to_idx(b):
            if colsplit:
                for g in range(_CH // 16):
                    sl = pl.ds(g * 16, 16)
                    lsrc[b][sl] = lsrc[b][sl] * 2 + c

        def gissue(b):
            pltpu.async_copy(table_hbm.at[lsrc[b]], rows[b], sem_g)

        def gwait(b):
            pltpu.make_async_copy(
                table_hbm.at[lsrc[b]], rows[b], sem_g).wait()

        def dstcopy(b):
            for g in range(_CH // 16):
                sl = pl.ds(g * 16, 16)
                sdst[b][sl] = ldst[b][sl]

        def sissue(b):
            pltpu.async_copy(rows[b], acc.at[sdst[b]], sem_s, add=True)

        def swait(b):
            pltpu.make_async_copy(rows[b], acc.at[sdst[b]], sem_s).wait()

        def body(j, b):
            gwait(b)
            dstcopy(b)
            sissue(b)

            @pl.when(j + _NB < nfull)
            def _():
                lissue(j + _NB, b)

            @pl.when(j > 0)
            def _():
                swait((b + _NB - 1) % _NB)

            @pl.when(j + _NB - 1 < nfull)
            def _():
                b3 = (b + _NB - 1) % _NB
                lwait(j + _NB - 1, b3)
                to_idx(b3)
                gissue(b3)

        for q in range(_NB - 1):
            lissue(q, q)
            lwait(q, q)
            to_idx(q)
            gissue(q)
        lissue(_NB - 1, _NB - 1)

        def group(jp, carry):
            j = _NB * jp
            for b in range(_NB):
                body(j + b, b)
            return carry

        lax.fori_loop(0, nfull // _NB, group, 0)
        swait((nfull - 1) % _NB)

        if tail:
            for g in range(_CH // 16):
                sl = pl.ds(g * 16, 16)
                lsrc[0][sl] = jnp.zeros((16,), jnp.int32)
                sdst[0][sl] = jnp.full((16,), _TRASH, jnp.int32)
            tb = ebase + nfull * _CH
            pltpu.sync_copy(src_hbm.at[pl.ds(tb, tail)],
                            lsrc[0].at[pl.ds(0, tail)])
            pltpu.sync_copy(dst_hbm.at[pl.ds(tb, tail)],
                            sdst[0].at[pl.ds(0, tail)])
            to_idx(0)
            pltpu.async_copy(table_hbm.at[lsrc[0]], rows[0], sem_g).wait()
            pltpu.sync_copy(rows[0], acc.at[sdst[0]], add=True)

        plsc.subcore_barrier()

        for kk in range(nzfull // _NS + 1):
            cid = s + kk * _NS

            @pl.when(cid < nzfull)
            def _():
                pltpu.sync_copy(acc.at[pl.ds(cid * _CH, _CH)],
                                out_hbm.at[c, pl.ds(cid * _CH, _CH)])

            @pl.when(cid == nzfull)
            def _():
                rem = _ACC_ROWS - nzfull * _CH
                pltpu.sync_copy(acc.at[pl.ds(nzfull * _CH, rem)],
                                out_hbm.at[c, pl.ds(nzfull * _CH, rem)])

    return k


_BN = 1000


def _mlp_body(colsplit, h_ref, a_ref, wa_ref, ba_ref, wb_ref, bb_ref, o_ref):
    if colsplit:
        agg = jnp.concatenate([a_ref[0], a_ref[1]], axis=1)
    else:
        agg = a_ref[0] + a_ref[1]
    z = h_ref[...] + agg
    t = jnp.maximum(
        jnp.dot(z, wa_ref[...], preferred_element_type=jnp.float32)
        + ba_ref[...], 0.0)
    o = (jnp.dot(t, wb_ref[...], preferred_element_type=jnp.float32)
         + bb_ref[...])
    o_ref[...] = jnp.maximum(o, 0.0)


def _mlp_tc(h, agg, Wa, ba, Wb, bb, colsplit):
    Din = h.shape[1]
    D2 = agg.shape[2]
    grid = (N_NODES // _BN,)
    return pl.pallas_call(
        functools.partial(_mlp_body, colsplit),
        grid=grid,
        in_specs=[
            pl.BlockSpec((_BN, Din), lambda i: (i, 0)),
            pl.BlockSpec((2, _BN, D2), lambda i: (0, i, 0)),
            pl.BlockSpec((Din, D_EMB), lambda i: (0, 0)),
            pl.BlockSpec((1, D_EMB), lambda i: (0, 0)),
            pl.BlockSpec((D_EMB, D_EMB), lambda i: (0, 0)),
            pl.BlockSpec((1, D_EMB), lambda i: (0, 0)),
        ],
        out_specs=pl.BlockSpec((_BN, D_EMB), lambda i: (i, 0)),
        out_shape=jax.ShapeDtypeStruct((N_NODES, D_EMB), jnp.float32),
    )(h, agg, Wa, ba.reshape(1, -1), Wb, bb.reshape(1, -1))


def _pool_body(ne_ref, batch_ref, o_ref, acc, cnt):
    i = pl.program_id(0)
    oh = (lax.broadcasted_iota(jnp.int32, (N_GRAPHS, _BN), 0)
          == batch_ref[0]).astype(jnp.float32)
    part = jnp.dot(oh, ne_ref[...], preferred_element_type=jnp.float32)
    pcnt = jnp.sum(oh, axis=1, keepdims=True)

    @pl.when(i == 0)
    def _():
        acc[...] = part
        cnt[...] = pcnt

    @pl.when(i > 0)
    def _():
        acc[...] += part
        cnt[...] += pcnt

    @pl.when(i == pl.num_programs(0) - 1)
    def _():
        o_ref[...] = acc[...] / jnp.maximum(cnt[...], 1.0)


def _pool_tc(h, batch3d):
    D = h.shape[1]
    return pl.pallas_call(
        _pool_body,
        grid=(N_NODES // _BN,),
        in_specs=[
            pl.BlockSpec((_BN, D), lambda i: (i, 0)),
            pl.BlockSpec((1, 1, _BN), lambda i: (i, 0, 0)),
        ],
        out_specs=pl.BlockSpec((N_GRAPHS, D), lambda i: (0, 0)),
        out_shape=jax.ShapeDtypeStruct((N_GRAPHS, D), jnp.float32),
        scratch_shapes=[
            pltpu.VMEM((N_GRAPHS, D), jnp.float32),
            pltpu.VMEM((N_GRAPHS, 1), jnp.float32),
        ],
    )(h, batch3d)


def kernel(x, edge_index, batch,
           W0a, b0a, W0b, b0b, W1a, b1a, W1b, b1b, W2a, b2a, W2b, b2b):
    params = [(W0a, b0a, W0b, b0b), (W1a, b1a, W1b, b1b),
              (W2a, b2a, W2b, b2b)]
    h = x
    hs = []
    pooled = []
    src = edge_index[0]
    dst = edge_index[1]
    batch3d = batch.reshape(N_NODES // _BN, 1, _BN)
    first = True
    for Wa, ba, Wb, bb in params:
        if first:
            agg = _seg_sum_sc(h.shape[1], False)(src, dst, h)
        else:
            D2 = h.shape[1] // 2
            table = h.reshape(2 * N_NODES, D2)
            agg = _seg_sum_sc(D2, True)(src, dst, table)
        h = _mlp_tc(h, agg, Wa, ba, Wb, bb, colsplit=not first)
        first = False
        hs.append(h)
        pooled.append(_pool_tc(h, batch3d))
    node_embed = jnp.concatenate(hs, axis=1)
    graph_embed = jnp.concatenate(pooled, axis=1)
    return graph_embed, node_embed


# speedup vs baseline: 1.3634x; 1.0245x over previous
import functools

import jax
import jax.numpy as jnp
from jax import lax
from jax.experimental import pallas as pl
from jax.experimental.pallas import tpu as pltpu
from jax.experimental.pallas import tpu_sc as plsc

N_NODES = 10000
N_EDGES = 320000
N_GRAPHS = 128
D_IN = 128
D_EMB = 256

_NS = 16
_CH = 96
_NB = 4
_ACC_ROWS = 10016
_TRASH = N_NODES


def _seg_sum_sc(D2, colsplit):
    mesh = plsc.VectorSubcoreMesh(core_axis_name="c", subcore_axis_name="s")
    ept = (N_EDGES if colsplit else N_EDGES // 2) // _NS
    nfull = ept // _CH
    tail = ept - nfull * _CH
    assert nfull % _NB == 0

    @functools.partial(
        pl.kernel,
        out_type=jax.ShapeDtypeStruct((2, _ACC_ROWS, D2), jnp.float32),
        mesh=mesh,
        scratch_types=[
            [pltpu.VMEM((_CH,), jnp.int32) for _ in range(_NB)],
            [pltpu.VMEM((_CH,), jnp.int32) for _ in range(_NB)],
            [pltpu.VMEM((_CH,), jnp.int32) for _ in range(_NB)],
            [pltpu.VMEM((_CH, D2), jnp.float32) for _ in range(_NB)],
            pltpu.VMEM_SHARED((_ACC_ROWS, D2), jnp.float32),
            pltpu.SemaphoreType.DMA,
            pltpu.SemaphoreType.DMA,
            pltpu.SemaphoreType.DMA,
        ],
    )
    def k(src_hbm, dst_hbm, table_hbm, out_hbm,
          lsrc, ldst, sdst, rows, acc, sem_l, sem_g, sem_s):
        c = lax.axis_index("c")
        s = lax.axis_index("s")
        if colsplit:
            ebase = s * ept
        else:
            ebase = c * (N_EDGES // 2) + s * ept

        zero16 = jnp.zeros((16,), jnp.float32)

        def zrow(r, carry):
            for g in range(D2 // 16):
                rows[0][r, pl.ds(g * 16, 16)] = zero16
            return carry

        lax.fori_loop(0, _CH, zrow, 0)
        nzfull = _ACC_ROWS // _CH
        for kk in range(nzfull // _NS + 1):
            cid = s + kk * _NS

            @pl.when(cid < nzfull)
            def _():
                pltpu.sync_copy(rows[0], acc.at[pl.ds(cid * _CH, _CH)])

            @pl.when(cid == nzfull)
            def _():
                pltpu.sync_copy(rows[0].at[pl.ds(0, _ACC_ROWS - nzfull * _CH)],
                                acc.at[pl.ds(nzfull * _CH,
                                             _ACC_ROWS - nzfull * _CH)])

        plsc.subcore_barrier()

        def lissue(j, b):
            base = ebase + j * _CH
            pltpu.async_copy(src_hbm.at[pl.ds(base, _CH)], lsrc[b], sem_l)
            pltpu.async_copy(dst_hbm.at[pl.ds(base, _CH)], ldst[b], sem_l)

        def lwait(j, b):
            base = ebase + j * _CH
            pltpu.make_async_copy(
                src_hbm.at[pl.ds(base, _CH)], lsrc[b], sem_l).wait()
            pltpu.make_async_copy(
                dst_hbm.at[pl.ds(base, _CH)], ldst[b], sem_l).wait()

        def to_idx(b):
            if colsplit:
                for g in range(_CH // 16):
                    sl = pl.ds(g * 16, 16)
                    lsrc[b][sl] = lsrc[b][sl] * 2 + c

        def gissue(b):
            pltpu.async_copy(table_hbm.at[lsrc[b]], rows[b], sem_g)

        def gwait(b):
            pltpu.make_async_copy(
                table_hbm.at[lsrc[b]], rows[b], sem_g).wait()

        def dstcopy(b):
            for g in range(_CH // 16):
                sl = pl.ds(g * 16, 16)
                sdst[b][sl] = ldst[b][sl]

        def sissue(b):
            pltpu.async_copy(rows[b], acc.at[sdst[b]], sem_s, add=True)

        def swait(b):
            pltpu.make_async_copy(rows[b], acc.at[sdst[b]], sem_s).wait()

        def body(j, b):
            gwait(b)
            dstcopy(b)
            sissue(b)

            @pl.when(j + _NB < nfull)
            def _():
                lissue(j + _NB, b)

            @pl.when(j > 0)
            def _():
                swait((b + _NB - 1) % _NB)

            @pl.when(j + _NB - 1 < nfull)
            def _():
                b3 = (b + _NB - 1) % _NB
                lwait(j + _NB - 1, b3)
                to_idx(b3)
                gissue(b3)

        for q in range(_NB - 1):
            lissue(q, q)
            lwait(q, q)
            to_idx(q)
            gissue(q)
        lissue(_NB - 1, _NB - 1)

        def group(jp, carry):
            j = _NB * jp
            for b in range(_NB):
                body(j + b, b)
            return carry

        lax.fori_loop(0, nfull // _NB, group, 0)
        swait((nfull - 1) % _NB)

        if tail:
            for g in range(_CH // 16):
                sl = pl.ds(g * 16, 16)
                lsrc[0][sl] = jnp.zeros((16,), jnp.int32)
                sdst[0][sl] = jnp.full((16,), _TRASH, jnp.int32)
            tb = ebase + nfull * _CH
            pltpu.sync_copy(src_hbm.at[pl.ds(tb, tail)],
                            lsrc[0].at[pl.ds(0, tail)])
            pltpu.sync_copy(dst_hbm.at[pl.ds(tb, tail)],
                            sdst[0].at[pl.ds(0, tail)])
            to_idx(0)
            pltpu.async_copy(table_hbm.at[lsrc[0]], rows[0], sem_g).wait()
            pltpu.sync_copy(rows[0], acc.at[sdst[0]], add=True)

        plsc.subcore_barrier()

        for kk in range(nzfull // _NS + 1):
            cid = s + kk * _NS

            @pl.when(cid < nzfull)
            def _():
                pltpu.sync_copy(acc.at[pl.ds(cid * _CH, _CH)],
                                out_hbm.at[c, pl.ds(cid * _CH, _CH)])

            @pl.when(cid == nzfull)
            def _():
                rem = _ACC_ROWS - nzfull * _CH
                pltpu.sync_copy(acc.at[pl.ds(nzfull * _CH, rem)],
                                out_hbm.at[c, pl.ds(nzfull * _CH, rem)])

    return k


_BN = 1000


def _mlp_body(colsplit, nprev,
              h_ref, a_ref, wa_ref, ba_ref, wb_ref, bb_ref, batch_ref,
              *refs):
    prev = refs[:nprev]
    o_ref, p_ref = refs[nprev], refs[nprev + 1]
    acc, cnt = refs[nprev + 2], refs[nprev + 3]
    i = pl.program_id(0)
    if colsplit:
        agg = jnp.concatenate([a_ref[0], a_ref[1]], axis=1)
    else:
        agg = a_ref[0] + a_ref[1]
    z = h_ref[...] + agg
    t = jnp.maximum(
        jnp.dot(z, wa_ref[...], preferred_element_type=jnp.float32)
        + ba_ref[...], 0.0)
    o = (jnp.dot(t, wb_ref[...], preferred_element_type=jnp.float32)
         + bb_ref[...])
    o = jnp.maximum(o, 0.0)
    if nprev:
        o_ref[...] = jnp.concatenate([p[...] for p in prev] + [o], axis=1)
    else:
        o_ref[...] = o

    oh = (lax.broadcasted_iota(jnp.int32, (N_GRAPHS, _BN), 0)
          == batch_ref[0]).astype(jnp.float32)
    part = jnp.dot(oh, o, preferred_element_type=jnp.float32)
    pcnt = jnp.sum(oh, axis=1, keepdims=True)

    @pl.when(i == 0)
    def _():
        acc[...] = part
        cnt[...] = pcnt

    @pl.when(i > 0)
    def _():
        acc[...] += part
        cnt[...] += pcnt

    @pl.when(i == pl.num_programs(0) - 1)
    def _():
        p_ref[...] = acc[...] / jnp.maximum(cnt[...], 1.0)


def _mlp_tc(h, agg, Wa, ba, Wb, bb, batch3d, colsplit, prev=()):
    Din = h.shape[1]
    D2 = agg.shape[2]
    dout = D_EMB * (1 + len(prev))
    return pl.pallas_call(
        functools.partial(_mlp_body, colsplit, len(prev)),
        grid=(N_NODES // _BN,),
        in_specs=[
            pl.BlockSpec((_BN, Din), lambda i: (i, 0)),
            pl.BlockSpec((2, _BN, D2), lambda i: (0, i, 0)),
            pl.BlockSpec((Din, D_EMB), lambda i: (0, 0)),
            pl.BlockSpec((1, D_EMB), lambda i: (0, 0)),
            pl.BlockSpec((D_EMB, D_EMB), lambda i: (0, 0)),
            pl.BlockSpec((1, D_EMB), lambda i: (0, 0)),
            pl.BlockSpec((1, 1, _BN), lambda i: (i, 0, 0)),
        ] + [pl.BlockSpec((_BN, D_EMB), lambda i: (i, 0)) for _ in prev],
        out_specs=[
            pl.BlockSpec((_BN, dout), lambda i: (i, 0)),
            pl.BlockSpec((N_GRAPHS, D_EMB), lambda i: (0, 0)),
        ],
        out_shape=[
            jax.ShapeDtypeStruct((N_NODES, dout), jnp.float32),
            jax.ShapeDtypeStruct((N_GRAPHS, D_EMB), jnp.float32),
        ],
        scratch_shapes=[
            pltpu.VMEM((N_GRAPHS, D_EMB), jnp.float32),
            pltpu.VMEM((N_GRAPHS, 1), jnp.float32),
        ],
    )(h, agg, Wa, ba.reshape(1, -1), Wb, bb.reshape(1, -1), batch3d, *prev)


def kernel(x, edge_index, batch,
           W0a, b0a, W0b, b0b, W1a, b1a, W1b, b1b, W2a, b2a, W2b, b2b):
    params = [(W0a, b0a, W0b, b0b), (W1a, b1a, W1b, b1b),
              (W2a, b2a, W2b, b2b)]
    src = edge_index[0]
    dst = edge_index[1]
    batch3d = batch.reshape(N_NODES // _BN, 1, _BN)
    h = x
    hs = []
    pooled = []
    for li, (Wa, ba, Wb, bb) in enumerate(params):
        if li == 0:
            agg = _seg_sum_sc(h.shape[1], False)(src, dst, h)
        else:
            D2 = h.shape[1] // 2
            table = h.reshape(2 * N_NODES, D2)
            agg = _seg_sum_sc(D2, True)(src, dst, table)
        prev = tuple(hs) if li == len(params) - 1 else ()
        h_out, p = _mlp_tc(h, agg, Wa, ba, Wb, bb, batch3d,
                           colsplit=li > 0, prev=prev)
        pooled.append(p)
        if prev:
            node_embed = h_out
        else:
            hs.append(h_out)
            h = h_out
    graph_embed = jnp.concatenate(pooled, axis=1)
    return graph_embed, node_embed


# speedup vs baseline: 1.5210x; 1.1156x over previous
import functools

import jax
import jax.numpy as jnp
from jax import lax
from jax.experimental import pallas as pl
from jax.experimental.pallas import tpu as pltpu
from jax.experimental.pallas import tpu_sc as plsc

N_NODES = 10000
N_EDGES = 320000
N_GRAPHS = 128
D_IN = 128
D_EMB = 256

_NS = 16
_CH = 64
_NB = 5
_ACC_ROWS = 10016
_TRASH = N_NODES


def _seg_sum_sc(D2, colsplit):
    mesh = plsc.VectorSubcoreMesh(core_axis_name="c", subcore_axis_name="s")
    ept = (N_EDGES if colsplit else N_EDGES // 2) // _NS
    nfull = ept // _CH
    tail = ept - nfull * _CH

    @functools.partial(
        pl.kernel,
        out_type=jax.ShapeDtypeStruct((2, _ACC_ROWS, D2), jnp.float32),
        mesh=mesh,
        scratch_types=[
            [pltpu.VMEM((_CH,), jnp.int32) for _ in range(_NB)],
            [pltpu.VMEM((_CH,), jnp.int32) for _ in range(_NB)],
            [pltpu.VMEM((_CH,), jnp.int32) for _ in range(_NB)],
            [pltpu.VMEM((_CH, D2), jnp.float32) for _ in range(_NB)],
            pltpu.VMEM_SHARED((_ACC_ROWS, D2), jnp.float32),
            pltpu.SemaphoreType.DMA,
            pltpu.SemaphoreType.DMA,
            pltpu.SemaphoreType.DMA,
        ],
    )
    def k(src_hbm, dst_hbm, table_hbm, out_hbm,
          lsrc, ldst, sdst, rows, acc, sem_l, sem_g, sem_s):
        c = lax.axis_index("c")
        s = lax.axis_index("s")
        if colsplit:
            ebase = s * ept
        else:
            ebase = c * (N_EDGES // 2) + s * ept

        zero16 = jnp.zeros((16,), jnp.float32)

        def zrow(r, carry):
            for g in range(D2 // 16):
                rows[0][r, pl.ds(g * 16, 16)] = zero16
            return carry

        lax.fori_loop(0, _CH, zrow, 0)
        nzfull = _ACC_ROWS // _CH
        for kk in range(nzfull // _NS + 1):
            cid = s + kk * _NS

            @pl.when(cid < nzfull)
            def _():
                pltpu.sync_copy(rows[0], acc.at[pl.ds(cid * _CH, _CH)])

            @pl.when(cid == nzfull)
            def _():
                pltpu.sync_copy(rows[0].at[pl.ds(0, _ACC_ROWS - nzfull * _CH)],
                                acc.at[pl.ds(nzfull * _CH,
                                             _ACC_ROWS - nzfull * _CH)])

        plsc.subcore_barrier()

        def lissue(j, b):
            base = ebase + j * _CH
            pltpu.async_copy(src_hbm.at[pl.ds(base, _CH)], lsrc[b], sem_l)
            pltpu.async_copy(dst_hbm.at[pl.ds(base, _CH)], ldst[b], sem_l)

        def lwait(j, b):
            base = ebase + j * _CH
            pltpu.make_async_copy(
                src_hbm.at[pl.ds(base, _CH)], lsrc[b], sem_l).wait()
            pltpu.make_async_copy(
                dst_hbm.at[pl.ds(base, _CH)], ldst[b], sem_l).wait()

        def to_idx(b):
            if colsplit:
                for g in range(_CH // 16):
                    sl = pl.ds(g * 16, 16)
                    lsrc[b][sl] = lsrc[b][sl] * 2 + c

        def gissue(b):
            pltpu.async_copy(table_hbm.at[lsrc[b]], rows[b], sem_g)

        def gwait(b):
            pltpu.make_async_copy(
                table_hbm.at[lsrc[b]], rows[b], sem_g).wait()

        def dstcopy(b):
            for g in range(_CH // 16):
                sl = pl.ds(g * 16, 16)
                sdst[b][sl] = ldst[b][sl]

        def sissue(b):
            pltpu.async_copy(rows[b], acc.at[sdst[b]], sem_s, add=True)

        def swait(b):
            pltpu.make_async_copy(rows[b], acc.at[sdst[b]], sem_s).wait()

        def body(j, b):
            gwait(b)
            dstcopy(b)
            sissue(b)

            @pl.when(j + _NB < nfull)
            def _():
                lissue(j + _NB, b)

            @pl.when(j > 0)
            def _():
                swait((b + _NB - 1) % _NB)

            @pl.when(j + _NB - 1 < nfull)
            def _():
                b3 = (b + _NB - 1) % _NB
                lwait(j + _NB - 1, b3)
                to_idx(b3)
                gissue(b3)

        for q in range(_NB - 1):
            lissue(q, q)
            lwait(q, q)
            to_idx(q)
            gissue(q)
        lissue(_NB - 1, _NB - 1)

        def group(jp, carry):
            j = _NB * jp
            for b in range(_NB):
                body(j + b, b)
            return carry

        lax.fori_loop(0, nfull // _NB, group, 0)
        for r in range(nfull % _NB):
            j = (nfull // _NB) * _NB + r
            body(j, j % _NB)
        swait((nfull - 1) % _NB)

        if tail:
            for g in range(_CH // 16):
                sl = pl.ds(g * 16, 16)
                lsrc[0][sl] = jnp.zeros((16,), jnp.int32)
                sdst[0][sl] = jnp.full((16,), _TRASH, jnp.int32)
            tb = ebase + nfull * _CH
            pltpu.sync_copy(src_hbm.at[pl.ds(tb, tail)],
                            lsrc[0].at[pl.ds(0, tail)])
            pltpu.sync_copy(dst_hbm.at[pl.ds(tb, tail)],
                            sdst[0].at[pl.ds(0, tail)])
            to_idx(0)
            pltpu.async_copy(table_hbm.at[lsrc[0]], rows[0], sem_g).wait()
            pltpu.sync_copy(rows[0], acc.at[sdst[0]], add=True)

        plsc.subcore_barrier()

        for kk in range(nzfull // _NS + 1):
            cid = s + kk * _NS

            @pl.when(cid < nzfull)
            def _():
                pltpu.sync_copy(acc.at[pl.ds(cid * _CH, _CH)],
                                out_hbm.at[c, pl.ds(cid * _CH, _CH)])

            @pl.when(cid == nzfull)
            def _():
                rem = _ACC_ROWS - nzfull * _CH
                pltpu.sync_copy(acc.at[pl.ds(nzfull * _CH, rem)],
                                out_hbm.at[c, pl.ds(nzfull * _CH, rem)])

    return k


_BN = 1000


def _mlp_body(colsplit, nprev,
              h_ref, a_ref, wa_ref, ba_ref, wb_ref, bb_ref, batch_ref,
              *refs):
    prev = refs[:nprev]
    o_ref, p_ref = refs[nprev], refs[nprev + 1]
    acc, cnt = refs[nprev + 2], refs[nprev + 3]
    i = pl.program_id(0)
    if colsplit:
        agg = jnp.concatenate([a_ref[0], a_ref[1]], axis=1)
    else:
        agg = a_ref[0] + a_ref[1]
    z = h_ref[...] + agg
    t = jnp.maximum(
        jnp.dot(z, wa_ref[...], preferred_element_type=jnp.float32)
        + ba_ref[...], 0.0)
    o = (jnp.dot(t, wb_ref[...], preferred_element_type=jnp.float32)
         + bb_ref[...])
    o = jnp.maximum(o, 0.0)
    if nprev:
        o_ref[...] = jnp.concatenate([p[...] for p in prev] + [o], axis=1)
    else:
        o_ref[...] = o

    oh = (lax.broadcasted_iota(jnp.int32, (N_GRAPHS, _BN), 0)
          == batch_ref[0]).astype(jnp.float32)
    part = jnp.dot(oh, o, preferred_element_type=jnp.float32)
    pcnt = jnp.sum(oh, axis=1, keepdims=True)

    @pl.when(i == 0)
    def _():
        acc[...] = part
        cnt[...] = pcnt

    @pl.when(i > 0)
    def _():
        acc[...] += part
        cnt[...] += pcnt

    @pl.when(i == pl.num_programs(0) - 1)
    def _():
        p_ref[...] = acc[...] / jnp.maximum(cnt[...], 1.0)


def _mlp_tc(h, agg, Wa, ba, Wb, bb, batch3d, colsplit, prev=()):
    Din = h.shape[1]
    D2 = agg.shape[2]
    dout = D_EMB * (1 + len(prev))
    return pl.pallas_call(
        functools.partial(_mlp_body, colsplit, len(prev)),
        grid=(N_NODES // _BN,),
        in_specs=[
            pl.BlockSpec((_BN, Din), lambda i: (i, 0)),
            pl.BlockSpec((2, _BN, D2), lambda i: (0, i, 0)),
            pl.BlockSpec((Din, D_EMB), lambda i: (0, 0)),
            pl.BlockSpec((1, D_EMB), lambda i: (0, 0)),
            pl.BlockSpec((D_EMB, D_EMB), lambda i: (0, 0)),
            pl.BlockSpec((1, D_EMB), lambda i: (0, 0)),
            pl.BlockSpec((1, 1, _BN), lambda i: (i, 0, 0)),
        ] + [pl.BlockSpec((_BN, D_EMB), lambda i: (i, 0)) for _ in prev],
        out_specs=[
            pl.BlockSpec((_BN, dout), lambda i: (i, 0)),
            pl.BlockSpec((N_GRAPHS, D_EMB), lambda i: (0, 0)),
        ],
        out_shape=[
            jax.ShapeDtypeStruct((N_NODES, dout), jnp.float32),
            jax.ShapeDtypeStruct((N_GRAPHS, D_EMB), jnp.float32),
        ],
        scratch_shapes=[
            pltpu.VMEM((N_GRAPHS, D_EMB), jnp.float32),
            pltpu.VMEM((N_GRAPHS, 1), jnp.float32),
        ],
    )(h, agg, Wa, ba.reshape(1, -1), Wb, bb.reshape(1, -1), batch3d, *prev)


def kernel(x, edge_index, batch,
           W0a, b0a, W0b, b0b, W1a, b1a, W1b, b1b, W2a, b2a, W2b, b2b):
    params = [(W0a, b0a, W0b, b0b), (W1a, b1a, W1b, b1b),
              (W2a, b2a, W2b, b2b)]
    src = edge_index[0]
    dst = edge_index[1]
    batch3d = batch.reshape(N_NODES // _BN, 1, _BN)
    h = x
    hs = []
    pooled = []
    for li, (Wa, ba, Wb, bb) in enumerate(params):
        if li == 0:
            agg = _seg_sum_sc(h.shape[1], False)(src, dst, h)
        else:
            D2 = h.shape[1] // 2
            table = h.reshape(2 * N_NODES, D2)
            agg = _seg_sum_sc(D2, True)(src, dst, table)
        prev = tuple(hs) if li == len(params) - 1 else ()
        h_out, p = _mlp_tc(h, agg, Wa, ba, Wb, bb, batch3d,
                           colsplit=li > 0, prev=prev)
        pooled.append(p)
        if prev:
            node_embed = h_out
        else:
            hs.append(h_out)
            h = h_out
    graph_embed = jnp.concatenate(pooled, axis=1)
    return graph_embed, node_embed


# speedup vs baseline: 1.5750x; 1.0355x over previous
import functools

import jax
import jax.numpy as jnp
from jax import lax
from jax.experimental import pallas as pl
from jax.experimental.pallas import tpu as pltpu
from jax.experimental.pallas import tpu_sc as plsc

N_NODES = 10000
N_EDGES = 320000
N_GRAPHS = 128
D_IN = 128
D_EMB = 256

_NS = 16
_CH = 48
_NB = 7
_ACC_ROWS = 10016
_TRASH = N_NODES


def _seg_sum_sc(D2, colsplit):
    mesh = plsc.VectorSubcoreMesh(core_axis_name="c", subcore_axis_name="s")
    ept = (N_EDGES if colsplit else N_EDGES // 2) // _NS
    nfull = ept // _CH
    tail = ept - nfull * _CH

    @functools.partial(
        pl.kernel,
        out_type=jax.ShapeDtypeStruct((2, _ACC_ROWS, D2), jnp.float32),
        mesh=mesh,
        scratch_types=[
            [pltpu.VMEM((_CH,), jnp.int32) for _ in range(_NB)],
            [pltpu.VMEM((_CH,), jnp.int32) for _ in range(_NB)],
            [pltpu.VMEM((_CH,), jnp.int32) for _ in range(_NB)],
            [pltpu.VMEM((_CH, D2), jnp.float32) for _ in range(_NB)],
            pltpu.VMEM_SHARED((_ACC_ROWS, D2), jnp.float32),
            pltpu.SemaphoreType.DMA,
            pltpu.SemaphoreType.DMA,
            pltpu.SemaphoreType.DMA,
        ],
    )
    def k(src_hbm, dst_hbm, table_hbm, out_hbm,
          lsrc, ldst, sdst, rows, acc, sem_l, sem_g, sem_s):
        c = lax.axis_index("c")
        s = lax.axis_index("s")
        if colsplit:
            ebase = s * ept
        else:
            ebase = c * (N_EDGES // 2) + s * ept

        zero16 = jnp.zeros((16,), jnp.float32)

        def zrow(r, carry):
            for g in range(D2 // 16):
                rows[0][r, pl.ds(g * 16, 16)] = zero16
            return carry

        lax.fori_loop(0, _CH, zrow, 0)
        nzfull = _ACC_ROWS // _CH
        for kk in range(nzfull // _NS + 1):
            cid = s + kk * _NS

            @pl.when(cid < nzfull)
            def _():
                pltpu.sync_copy(rows[0], acc.at[pl.ds(cid * _CH, _CH)])

            @pl.when(cid == nzfull)
            def _():
                pltpu.sync_copy(rows[0].at[pl.ds(0, _ACC_ROWS - nzfull * _CH)],
                                acc.at[pl.ds(nzfull * _CH,
                                             _ACC_ROWS - nzfull * _CH)])

        plsc.subcore_barrier()

        def lissue(j, b):
            base = ebase + j * _CH
            pltpu.async_copy(src_hbm.at[pl.ds(base, _CH)], lsrc[b], sem_l)
            pltpu.async_copy(dst_hbm.at[pl.ds(base, _CH)], ldst[b], sem_l)

        def lwait(j, b):
            base = ebase + j * _CH
            pltpu.make_async_copy(
                src_hbm.at[pl.ds(base, _CH)], lsrc[b], sem_l).wait()
            pltpu.make_async_copy(
                dst_hbm.at[pl.ds(base, _CH)], ldst[b], sem_l).wait()

        def to_idx(b):
            if colsplit:
                for g in range(_CH // 16):
                    sl = pl.ds(g * 16, 16)
                    lsrc[b][sl] = lsrc[b][sl] * 2 + c

        def gissue(b):
            pltpu.async_copy(table_hbm.at[lsrc[b]], rows[b], sem_g)

        def gwait(b):
            pltpu.make_async_copy(
                table_hbm.at[lsrc[b]], rows[b], sem_g).wait()

        def dstcopy(b):
            for g in range(_CH // 16):
                sl = pl.ds(g * 16, 16)
                sdst[b][sl] = ldst[b][sl]

        def sissue(b):
            pltpu.async_copy(rows[b], acc.at[sdst[b]], sem_s, add=True)

        def swait(b):
            pltpu.make_async_copy(rows[b], acc.at[sdst[b]], sem_s).wait()

        def body(j, b):
            gwait(b)
            dstcopy(b)
            sissue(b)

            @pl.when(j + _NB < nfull)
            def _():
                lissue(j + _NB, b)

            @pl.when(j > 0)
            def _():
                swait((b + _NB - 1) % _NB)

            @pl.when(j + _NB - 1 < nfull)
            def _():
                b3 = (b + _NB - 1) % _NB
                lwait(j + _NB - 1, b3)
                to_idx(b3)
                gissue(b3)

        for q in range(_NB - 1):
            lissue(q, q)
            lwait(q, q)
            to_idx(q)
            gissue(q)
        lissue(_NB - 1, _NB - 1)

        def group(jp, carry):
            j = _NB * jp
            for b in range(_NB):
                body(j + b, b)
            return carry

        lax.fori_loop(0, nfull // _NB, group, 0)
        for r in range(nfull % _NB):
            j = (nfull // _NB) * _NB + r
            body(j, j % _NB)
        swait((nfull - 1) % _NB)

        if tail:
            for g in range(_CH // 16):
                sl = pl.ds(g * 16, 16)
                lsrc[0][sl] = jnp.zeros((16,), jnp.int32)
                sdst[0][sl] = jnp.full((16,), _TRASH, jnp.int32)
            tb = ebase + nfull * _CH
            pltpu.sync_copy(src_hbm.at[pl.ds(tb, tail)],
                            lsrc[0].at[pl.ds(0, tail)])
            pltpu.sync_copy(dst_hbm.at[pl.ds(tb, tail)],
                            sdst[0].at[pl.ds(0, tail)])
            to_idx(0)
            pltpu.async_copy(table_hbm.at[lsrc[0]], rows[0], sem_g).wait()
            pltpu.sync_copy(rows[0], acc.at[sdst[0]], add=True)

        plsc.subcore_barrier()

        for kk in range(nzfull // _NS + 1):
            cid = s + kk * _NS

            @pl.when(cid < nzfull)
            def _():
                pltpu.sync_copy(acc.at[pl.ds(cid * _CH, _CH)],
                                out_hbm.at[c, pl.ds(cid * _CH, _CH)])

            @pl.when(cid == nzfull)
            def _():
                rem = _ACC_ROWS - nzfull * _CH
                pltpu.sync_copy(acc.at[pl.ds(nzfull * _CH, rem)],
                                out_hbm.at[c, pl.ds(nzfull * _CH, rem)])

    return k


_BN = 1000


def _mlp_body(colsplit, nprev,
              h_ref, a_ref, wa_ref, ba_ref, wb_ref, bb_ref, batch_ref,
              *refs):
    prev = refs[:nprev]
    o_ref, p_ref = refs[nprev], refs[nprev + 1]
    acc, cnt = refs[nprev + 2], refs[nprev + 3]
    i = pl.program_id(0)
    if colsplit:
        agg = jnp.concatenate([a_ref[0], a_ref[1]], axis=1)
    else:
        agg = a_ref[0] + a_ref[1]
    z = h_ref[...] + agg
    t = jnp.maximum(
        jnp.dot(z, wa_ref[...], preferred_element_type=jnp.float32)
        + ba_ref[...], 0.0)
    o = (jnp.dot(t, wb_ref[...], preferred_element_type=jnp.float32)
         + bb_ref[...])
    o = jnp.maximum(o, 0.0)
    if nprev:
        o_ref[...] = jnp.concatenate([p[...] for p in prev] + [o], axis=1)
    else:
        o_ref[...] = o

    oh = (lax.broadcasted_iota(jnp.int32, (N_GRAPHS, _BN), 0)
          == batch_ref[0]).astype(jnp.float32)
    part = jnp.dot(oh, o, preferred_element_type=jnp.float32)
    pcnt = jnp.sum(oh, axis=1, keepdims=True)

    @pl.when(i == 0)
    def _():
        acc[...] = part
        cnt[...] = pcnt

    @pl.when(i > 0)
    def _():
        acc[...] += part
        cnt[...] += pcnt

    @pl.when(i == pl.num_programs(0) - 1)
    def _():
        p_ref[...] = acc[...] / jnp.maximum(cnt[...], 1.0)


def _mlp_tc(h, agg, Wa, ba, Wb, bb, batch3d, colsplit, prev=()):
    Din = h.shape[1]
    D2 = agg.shape[2]
    dout = D_EMB * (1 + len(prev))
    return pl.pallas_call(
        functools.partial(_mlp_body, colsplit, len(prev)),
        grid=(N_NODES // _BN,),
        in_specs=[
            pl.BlockSpec((_BN, Din), lambda i: (i, 0)),
            pl.BlockSpec((2, _BN, D2), lambda i: (0, i, 0)),
            pl.BlockSpec((Din, D_EMB), lambda i: (0, 0)),
            pl.BlockSpec((1, D_EMB), lambda i: (0, 0)),
            pl.BlockSpec((D_EMB, D_EMB), lambda i: (0, 0)),
            pl.BlockSpec((1, D_EMB), lambda i: (0, 0)),
            pl.BlockSpec((1, 1, _BN), lambda i: (i, 0, 0)),
        ] + [pl.BlockSpec((_BN, D_EMB), lambda i: (i, 0)) for _ in prev],
        out_specs=[
            pl.BlockSpec((_BN, dout), lambda i: (i, 0)),
            pl.BlockSpec((N_GRAPHS, D_EMB), lambda i: (0, 0)),
        ],
        out_shape=[
            jax.ShapeDtypeStruct((N_NODES, dout), jnp.float32),
            jax.ShapeDtypeStruct((N_GRAPHS, D_EMB), jnp.float32),
        ],
        scratch_shapes=[
            pltpu.VMEM((N_GRAPHS, D_EMB), jnp.float32),
            pltpu.VMEM((N_GRAPHS, 1), jnp.float32),
        ],
    )(h, agg, Wa, ba.reshape(1, -1), Wb, bb.reshape(1, -1), batch3d, *prev)


def kernel(x, edge_index, batch,
           W0a, b0a, W0b, b0b, W1a, b1a, W1b, b1b, W2a, b2a, W2b, b2b):
    params = [(W0a, b0a, W0b, b0b), (W1a, b1a, W1b, b1b),
              (W2a, b2a, W2b, b2b)]
    src = edge_index[0]
    dst = edge_index[1]
    batch3d = batch.reshape(N_NODES // _BN, 1, _BN)
    h = x
    hs = []
    pooled = []
    for li, (Wa, ba, Wb, bb) in enumerate(params):
        if li == 0:
            agg = _seg_sum_sc(h.shape[1], False)(src, dst, h)
        else:
            D2 = h.shape[1] // 2
            table = h.reshape(2 * N_NODES, D2)
            agg = _seg_sum_sc(D2, True)(src, dst, table)
        prev = tuple(hs) if li == len(params) - 1 else ()
        h_out, p = _mlp_tc(h, agg, Wa, ba, Wb, bb, batch3d,
                           colsplit=li > 0, prev=prev)
        pooled.append(p)
        if prev:
            node_embed = h_out
        else:
            hs.append(h_out)
            h = h_out
    graph_embed = jnp.concatenate(pooled, axis=1)
    return graph_embed, node_embed


# speedup vs baseline: 1.5755x; 1.0003x over previous
import functools

import jax
import jax.numpy as jnp
from jax import lax
from jax.experimental import pallas as pl
from jax.experimental.pallas import tpu as pltpu
from jax.experimental.pallas import tpu_sc as plsc

N_NODES = 10000
N_EDGES = 320000
N_GRAPHS = 128
D_IN = 128
D_EMB = 256

_NS = 16
_CH = 48
_NB = 7
_ACC_ROWS = 10016
_TRASH = N_NODES


def _seg_sum_sc(D2, colsplit):
    mesh = plsc.VectorSubcoreMesh(core_axis_name="c", subcore_axis_name="s")
    ept = (N_EDGES if colsplit else N_EDGES // 2) // _NS
    nfull = ept // _CH
    tail = ept - nfull * _CH

    @functools.partial(
        pl.kernel,
        out_type=jax.ShapeDtypeStruct((2, _ACC_ROWS, D2), jnp.float32),
        mesh=mesh,
        scratch_types=[
            [pltpu.VMEM((_CH,), jnp.int32) for _ in range(_NB)],
            [pltpu.VMEM((_CH,), jnp.int32) for _ in range(_NB)],
            [pltpu.VMEM((_CH,), jnp.int32) for _ in range(_NB)],
            [pltpu.VMEM((_CH, D2), jnp.float32) for _ in range(_NB)],
            pltpu.VMEM_SHARED((_ACC_ROWS, D2), jnp.float32),
            pltpu.SemaphoreType.DMA,
            pltpu.SemaphoreType.DMA,
            pltpu.SemaphoreType.DMA,
        ],
    )
    def k(src_hbm, dst_hbm, table_hbm, out_hbm,
          lsrc, ldst, sdst, rows, acc, sem_l, sem_g, sem_s):
        c = lax.axis_index("c")
        s = lax.axis_index("s")
        if colsplit:
            ebase = s * ept
        else:
            ebase = c * (N_EDGES // 2) + s * ept

        zero16 = jnp.zeros((16,), jnp.float32)

        def zrow(r, carry):
            for g in range(D2 // 16):
                rows[0][r, pl.ds(g * 16, 16)] = zero16
            return carry

        lax.fori_loop(0, _CH, zrow, 0)
        nzfull = _ACC_ROWS // _CH
        for kk in range(nzfull // _NS + 1):
            cid = s + kk * _NS

            @pl.when(cid < nzfull)
            def _():
                pltpu.sync_copy(rows[0], acc.at[pl.ds(cid * _CH, _CH)])

            @pl.when(cid == nzfull)
            def _():
                pltpu.sync_copy(rows[0].at[pl.ds(0, _ACC_ROWS - nzfull * _CH)],
                                acc.at[pl.ds(nzfull * _CH,
                                             _ACC_ROWS - nzfull * _CH)])

        plsc.subcore_barrier()

        def lissue(j, b):
            base = ebase + j * _CH
            pltpu.async_copy(src_hbm.at[pl.ds(base, _CH)], lsrc[b], sem_l)
            pltpu.async_copy(dst_hbm.at[pl.ds(base, _CH)], ldst[b], sem_l)

        def lwait(j, b):
            base = ebase + j * _CH
            pltpu.make_async_copy(
                src_hbm.at[pl.ds(base, _CH)], lsrc[b], sem_l).wait()
            pltpu.make_async_copy(
                dst_hbm.at[pl.ds(base, _CH)], ldst[b], sem_l).wait()

        def to_idx(b):
            if colsplit:
                for g in range(_CH // 16):
                    sl = pl.ds(g * 16, 16)
                    lsrc[b][sl] = lsrc[b][sl] * 2 + c

        def gissue(b):
            pltpu.async_copy(table_hbm.at[lsrc[b]], rows[b], sem_g)

        def gwait(b):
            pltpu.make_async_copy(
                table_hbm.at[lsrc[b]], rows[b], sem_g).wait()

        def dstcopy(b):
            for g in range(_CH // 16):
                sl = pl.ds(g * 16, 16)
                sdst[b][sl] = ldst[b][sl]

        def sissue(b):
            pltpu.async_copy(rows[b], acc.at[sdst[b]], sem_s, add=True)

        def swait(b):
            pltpu.make_async_copy(rows[b], acc.at[sdst[b]], sem_s).wait()

        def body(j, b):
            gwait(b)
            dstcopy(b)
            sissue(b)

            @pl.when(j + _NB < nfull)
            def _():
                lissue(j + _NB, b)

            @pl.when(j > 0)
            def _():
                swait((b + _NB - 1) % _NB)

            @pl.when(j + _NB - 1 < nfull)
            def _():
                b3 = (b + _NB - 1) % _NB
                lwait(j + _NB - 1, b3)
                to_idx(b3)
                gissue(b3)

        for q in range(_NB - 1):
            lissue(q, q)
            lwait(q, q)
            to_idx(q)
            gissue(q)
        lissue(_NB - 1, _NB - 1)

        def group(jp, carry):
            j = _NB * jp
            for b in range(_NB):
                body(j + b, b)
            return carry

        lax.fori_loop(0, nfull // _NB, group, 0)
        for r in range(nfull % _NB):
            j = (nfull // _NB) * _NB + r
            body(j, j % _NB)
        swait((nfull - 1) % _NB)

        if tail:
            for g in range(_CH // 16):
                sl = pl.ds(g * 16, 16)
                lsrc[0][sl] = jnp.zeros((16,), jnp.int32)
                sdst[0][sl] = jnp.full((16,), _TRASH, jnp.int32)
            tb = ebase + nfull * _CH
            pltpu.sync_copy(src_hbm.at[pl.ds(tb, tail)],
                            lsrc[0].at[pl.ds(0, tail)])
            pltpu.sync_copy(dst_hbm.at[pl.ds(tb, tail)],
                            sdst[0].at[pl.ds(0, tail)])
            to_idx(0)
            pltpu.async_copy(table_hbm.at[lsrc[0]], rows[0], sem_g).wait()
            pltpu.sync_copy(rows[0], acc.at[sdst[0]], add=True)

        plsc.subcore_barrier()

        for kk in range(nzfull // _NS + 1):
            cid = s + kk * _NS

            @pl.when(cid < nzfull)
            def _():
                pltpu.sync_copy(acc.at[pl.ds(cid * _CH, _CH)],
                                out_hbm.at[c, pl.ds(cid * _CH, _CH)])

            @pl.when(cid == nzfull)
            def _():
                rem = _ACC_ROWS - nzfull * _CH
                pltpu.sync_copy(acc.at[pl.ds(nzfull * _CH, rem)],
                                out_hbm.at[c, pl.ds(nzfull * _CH, rem)])

    return k


_BN = 1000


def _mlp_body(colsplit, nprev,
              h_ref, a_ref, wa_ref, ba_ref, wb_ref, bb_ref, batch_ref,
              *refs):
    prev = refs[:nprev]
    o_ref, p_ref = refs[nprev], refs[nprev + 1]
    acc, cnt = refs[nprev + 2], refs[nprev + 3]
    i = pl.program_id(0)
    if colsplit:
        agg = jnp.concatenate([a_ref[0], a_ref[1]], axis=1)
    else:
        agg = a_ref[0] + a_ref[1]
    z = h_ref[...] + agg
    t = jnp.maximum(
        jnp.dot(z.astype(jnp.bfloat16), wa_ref[...].astype(jnp.bfloat16),
                preferred_element_type=jnp.float32)
        + ba_ref[...], 0.0)
    o = (jnp.dot(t.astype(jnp.bfloat16), wb_ref[...].astype(jnp.bfloat16),
                 preferred_element_type=jnp.float32)
         + bb_ref[...])
    o = jnp.maximum(o, 0.0)
    if nprev:
        o_ref[...] = jnp.concatenate([p[...] for p in prev] + [o], axis=1)
    else:
        o_ref[...] = o

    oh = (lax.broadcasted_iota(jnp.int32, (N_GRAPHS, _BN), 0)
          == batch_ref[0]).astype(jnp.float32)
    part = jnp.dot(oh, o, preferred_element_type=jnp.float32)
    pcnt = jnp.sum(oh, axis=1, keepdims=True)

    @pl.when(i == 0)
    def _():
        acc[...] = part
        cnt[...] = pcnt

    @pl.when(i > 0)
    def _():
        acc[...] += part
        cnt[...] += pcnt

    @pl.when(i == pl.num_programs(0) - 1)
    def _():
        p_ref[...] = acc[...] / jnp.maximum(cnt[...], 1.0)


def _mlp_tc(h, agg, Wa, ba, Wb, bb, batch3d, colsplit, prev=()):
    Din = h.shape[1]
    D2 = agg.shape[2]
    dout = D_EMB * (1 + len(prev))
    return pl.pallas_call(
        functools.partial(_mlp_body, colsplit, len(prev)),
        grid=(N_NODES // _BN,),
        in_specs=[
            pl.BlockSpec((_BN, Din), lambda i: (i, 0)),
            pl.BlockSpec((2, _BN, D2), lambda i: (0, i, 0)),
            pl.BlockSpec((Din, D_EMB), lambda i: (0, 0)),
            pl.BlockSpec((1, D_EMB), lambda i: (0, 0)),
            pl.BlockSpec((D_EMB, D_EMB), lambda i: (0, 0)),
            pl.BlockSpec((1, D_EMB), lambda i: (0, 0)),
            pl.BlockSpec((1, 1, _BN), lambda i: (i, 0, 0)),
        ] + [pl.BlockSpec((_BN, D_EMB), lambda i: (i, 0)) for _ in prev],
        out_specs=[
            pl.BlockSpec((_BN, dout), lambda i: (i, 0)),
            pl.BlockSpec((N_GRAPHS, D_EMB), lambda i: (0, 0)),
        ],
        out_shape=[
            jax.ShapeDtypeStruct((N_NODES, dout), jnp.float32),
            jax.ShapeDtypeStruct((N_GRAPHS, D_EMB), jnp.float32),
        ],
        scratch_shapes=[
            pltpu.VMEM((N_GRAPHS, D_EMB), jnp.float32),
            pltpu.VMEM((N_GRAPHS, 1), jnp.float32),
        ],
    )(h, agg, Wa, ba.reshape(1, -1), Wb, bb.reshape(1, -1), batch3d, *prev)


def kernel(x, edge_index, batch,
           W0a, b0a, W0b, b0b, W1a, b1a, W1b, b1b, W2a, b2a, W2b, b2b):
    params = [(W0a, b0a, W0b, b0b), (W1a, b1a, W1b, b1b),
              (W2a, b2a, W2b, b2b)]
    src = edge_index[0]
    dst = edge_index[1]
    batch3d = batch.reshape(N_NODES // _BN, 1, _BN)
    h = x
    hs = []
    pooled = []
    for li, (Wa, ba, Wb, bb) in enumerate(params):
        if li == 0:
            agg = _seg_sum_sc(h.shape[1], False)(src, dst, h)
        else:
            D2 = h.shape[1] // 2
            table = h.reshape(2 * N_NODES, D2)
            agg = _seg_sum_sc(D2, True)(src, dst, table)
        prev = tuple(hs) if li == len(params) - 1 else ()
        h_out, p = _mlp_tc(h, agg, Wa, ba, Wb, bb, batch3d,
                           colsplit=li > 0, prev=prev)
        pooled.append(p)
        if prev:
            node_embed = h_out
        else:
            hs.append(h_out)
            h = h_out
    graph_embed = jnp.concatenate(pooled, axis=1)
    return graph_embed, node_embed


# speedup vs baseline: 1.5767x; 1.0008x over previous
import functools

import jax
import jax.numpy as jnp
from jax import lax
from jax.experimental import pallas as pl
from jax.experimental.pallas import tpu as pltpu
from jax.experimental.pallas import tpu_sc as plsc

N_NODES = 10000
N_EDGES = 320000
N_GRAPHS = 128
D_IN = 128
D_EMB = 256

_NS = 16
_CH = 48
_NB = 7
_ACC_ROWS = 10016
_TRASH = N_NODES


def _seg_sum_sc(D2, colsplit):
    mesh = plsc.VectorSubcoreMesh(core_axis_name="c", subcore_axis_name="s")
    ept = (N_EDGES if colsplit else N_EDGES // 2) // _NS
    nfull = ept // _CH
    tail = ept - nfull * _CH

    @functools.partial(
        pl.kernel,
        out_type=jax.ShapeDtypeStruct((2, _ACC_ROWS, D2), jnp.float32),
        mesh=mesh,
        scratch_types=[
            [pltpu.VMEM((_CH,), jnp.int32) for _ in range(_NB)],
            [pltpu.VMEM((_CH,), jnp.int32) for _ in range(_NB)],
            [pltpu.VMEM((_CH,), jnp.int32) for _ in range(_NB)],
            [pltpu.VMEM((_CH, D2), jnp.float32) for _ in range(_NB)],
            pltpu.VMEM_SHARED((_ACC_ROWS, D2), jnp.float32),
            pltpu.SemaphoreType.DMA,
            pltpu.SemaphoreType.DMA,
            pltpu.SemaphoreType.DMA,
        ],
    )
    def k(src_hbm, dst_hbm, table_hbm, out_hbm,
          lsrc, ldst, sdst, rows, acc, sem_l, sem_g, sem_s):
        c = lax.axis_index("c")
        s = lax.axis_index("s")
        if colsplit:
            ebase = s * ept
        else:
            ebase = c * (N_EDGES // 2) + s * ept

        zero16 = jnp.zeros((16,), jnp.float32)

        def zrow(r, carry):
            for g in range(D2 // 16):
                rows[0][r, pl.ds(g * 16, 16)] = zero16
            return carry

        lax.fori_loop(0, _CH, zrow, 0)
        nzfull = _ACC_ROWS // _CH
        for kk in range(nzfull // _NS + 1):
            cid = s + kk * _NS

            @pl.when(cid < nzfull)
            def _():
                pltpu.sync_copy(rows[0], acc.at[pl.ds(cid * _CH, _CH)])

            @pl.when(cid == nzfull)
            def _():
                pltpu.sync_copy(rows[0].at[pl.ds(0, _ACC_ROWS - nzfull * _CH)],
                                acc.at[pl.ds(nzfull * _CH,
                                             _ACC_ROWS - nzfull * _CH)])

        plsc.subcore_barrier()

        def lissue(j, b):
            base = ebase + j * _CH
            pltpu.async_copy(src_hbm.at[pl.ds(base, _CH)], lsrc[b], sem_l)
            pltpu.async_copy(dst_hbm.at[pl.ds(base, _CH)], ldst[b], sem_l)

        def lwait(j, b):
            base = ebase + j * _CH
            pltpu.make_async_copy(
                src_hbm.at[pl.ds(base, _CH)], lsrc[b], sem_l).wait()
            pltpu.make_async_copy(
                dst_hbm.at[pl.ds(base, _CH)], ldst[b], sem_l).wait()

        def to_idx(b):
            if colsplit:
                for g in range(_CH // 16):
                    sl = pl.ds(g * 16, 16)
                    lsrc[b][sl] = lsrc[b][sl] * 2 + c

        def gissue(b):
            pltpu.async_copy(table_hbm.at[lsrc[b]], rows[b], sem_g)

        def gwait(b):
            pltpu.make_async_copy(
                table_hbm.at[lsrc[b]], rows[b], sem_g).wait()

        def dstcopy(b):
            for g in range(_CH // 16):
                sl = pl.ds(g * 16, 16)
                sdst[b][sl] = ldst[b][sl]

        def sissue(b):
            pltpu.async_copy(rows[b], acc.at[sdst[b]], sem_s, add=True)

        def swait(b):
            pltpu.make_async_copy(rows[b], acc.at[sdst[b]], sem_s).wait()

        def body(j, b):
            gwait(b)
            dstcopy(b)
            sissue(b)

            @pl.when(j + _NB < nfull)
            def _():
                lissue(j + _NB, b)

            @pl.when(j > 0)
            def _():
                swait((b + _NB - 1) % _NB)

            @pl.when(j + _NB - 1 < nfull)
            def _():
                b3 = (b + _NB - 1) % _NB
                lwait(j + _NB - 1, b3)
                to_idx(b3)
                gissue(b3)

        for q in range(_NB - 1):
            lissue(q, q)
            lwait(q, q)
            to_idx(q)
            gissue(q)
        lissue(_NB - 1, _NB - 1)

        def group(jp, carry):
            j = _NB * jp
            for b in range(_NB):
                body(j + b, b)
            return carry

        lax.fori_loop(0, nfull // _NB, group, 0)
        for r in range(nfull % _NB):
            j = (nfull // _NB) * _NB + r
            body(j, j % _NB)
        swait((nfull - 1) % _NB)

        if tail:
            for g in range(_CH // 16):
                sl = pl.ds(g * 16, 16)
                lsrc[0][sl] = jnp.zeros((16,), jnp.int32)
                sdst[0][sl] = jnp.full((16,), _TRASH, jnp.int32)
            tb = ebase + nfull * _CH
            pltpu.sync_copy(src_hbm.at[pl.ds(tb, tail)],
                            lsrc[0].at[pl.ds(0, tail)])
            pltpu.sync_copy(dst_hbm.at[pl.ds(tb, tail)],
                            sdst[0].at[pl.ds(0, tail)])
            to_idx(0)
            pltpu.async_copy(table_hbm.at[lsrc[0]], rows[0], sem_g).wait()
            pltpu.sync_copy(rows[0], acc.at[sdst[0]], add=True)

        plsc.subcore_barrier()

        for kk in range(nzfull // _NS + 1):
            cid = s + kk * _NS

            @pl.when(cid < nzfull)
            def _():
                pltpu.sync_copy(acc.at[pl.ds(cid * _CH, _CH)],
                                out_hbm.at[c, pl.ds(cid * _CH, _CH)])

            @pl.when(cid == nzfull)
            def _():
                rem = _ACC_ROWS - nzfull * _CH
                pltpu.sync_copy(acc.at[pl.ds(nzfull * _CH, rem)],
                                out_hbm.at[c, pl.ds(nzfull * _CH, rem)])

    return k


_BN = 1000


def _mlp_body(colsplit, nprev,
              h_ref, a_ref, wa_ref, ba_ref, wb_ref, bb_ref, batch_ref,
              *refs):
    prev = refs[:nprev]
    o_ref, p_ref = refs[nprev], refs[nprev + 1]
    acc, cnt = refs[nprev + 2], refs[nprev + 3]
    i = pl.program_id(0)
    if colsplit:
        agg = jnp.concatenate([a_ref[0], a_ref[1]], axis=1)
    else:
        agg = a_ref[0] + a_ref[1]
    z = h_ref[...] + agg
    t = jnp.maximum(
        jnp.dot(z, wa_ref[...], preferred_element_type=jnp.float32)
        + ba_ref[...], 0.0)
    o = (jnp.dot(t, wb_ref[...], preferred_element_type=jnp.float32)
         + bb_ref[...])
    o = jnp.maximum(o, 0.0)
    if nprev:
        o_ref[...] = jnp.concatenate([p[...] for p in prev] + [o], axis=1)
    else:
        o_ref[...] = o

    oh = (lax.broadcasted_iota(jnp.int32, (N_GRAPHS, _BN), 0)
          == batch_ref[0]).astype(jnp.float32)
    part = jnp.dot(oh, o, preferred_element_type=jnp.float32)
    pcnt = jnp.sum(oh, axis=1, keepdims=True)

    @pl.when(i == 0)
    def _():
        acc[...] = part
        cnt[...] = pcnt

    @pl.when(i > 0)
    def _():
        acc[...] += part
        cnt[...] += pcnt

    @pl.when(i == pl.num_programs(0) - 1)
    def _():
        p_ref[...] = acc[...] / jnp.maximum(cnt[...], 1.0)


def _mlp_tc(h, agg, Wa, ba, Wb, bb, batch3d, colsplit, prev=()):
    Din = h.shape[1]
    D2 = agg.shape[2]
    dout = D_EMB * (1 + len(prev))
    return pl.pallas_call(
        functools.partial(_mlp_body, colsplit, len(prev)),
        grid=(N_NODES // _BN,),
        in_specs=[
            pl.BlockSpec((_BN, Din), lambda i: (i, 0)),
            pl.BlockSpec((2, _BN, D2), lambda i: (0, i, 0)),
            pl.BlockSpec((Din, D_EMB), lambda i: (0, 0)),
            pl.BlockSpec((1, D_EMB), lambda i: (0, 0)),
            pl.BlockSpec((D_EMB, D_EMB), lambda i: (0, 0)),
            pl.BlockSpec((1, D_EMB), lambda i: (0, 0)),
            pl.BlockSpec((1, 1, _BN), lambda i: (i, 0, 0)),
        ] + [pl.BlockSpec((_BN, D_EMB), lambda i: (i, 0)) for _ in prev],
        out_specs=[
            pl.BlockSpec((_BN, dout), lambda i: (i, 0)),
            pl.BlockSpec((N_GRAPHS, D_EMB), lambda i: (0, 0)),
        ],
        out_shape=[
            jax.ShapeDtypeStruct((N_NODES, dout), jnp.float32),
            jax.ShapeDtypeStruct((N_GRAPHS, D_EMB), jnp.float32),
        ],
        scratch_shapes=[
            pltpu.VMEM((N_GRAPHS, D_EMB), jnp.float32),
            pltpu.VMEM((N_GRAPHS, 1), jnp.float32),
        ],
    )(h, agg, Wa, ba.reshape(1, -1), Wb, bb.reshape(1, -1), batch3d, *prev)


def kernel(x, edge_index, batch,
           W0a, b0a, W0b, b0b, W1a, b1a, W1b, b1b, W2a, b2a, W2b, b2b):
    params = [(W0a, b0a, W0b, b0b), (W1a, b1a, W1b, b1b),
              (W2a, b2a, W2b, b2b)]
    src = edge_index[0]
    dst = edge_index[1]
    batch3d = batch.reshape(N_NODES // _BN, 1, _BN)
    h = x
    hs = []
    pooled = []
    for li, (Wa, ba, Wb, bb) in enumerate(params):
        if li == 0:
            agg = _seg_sum_sc(h.shape[1], False)(src, dst, h)
        else:
            D2 = h.shape[1] // 2
            table = h.reshape(2 * N_NODES, D2)
            agg = _seg_sum_sc(D2, True)(src, dst, table)
        prev = tuple(hs) if li == len(params) - 1 else ()
        h_out, p = _mlp_tc(h, agg, Wa, ba, Wb, bb, batch3d,
                           colsplit=li > 0, prev=prev)
        pooled.append(p)
        if prev:
            node_embed = h_out
        else:
            hs.append(h_out)
            h = h_out
    graph_embed = jnp.concatenate(pooled, axis=1)
    return graph_embed, node_embed
